# Initial kernel scaffold; baseline (speedup 1.0000x reference)
import jax
import jax.numpy as jnp
from jax.experimental import pallas as pl


def kernel(x):
    raise NotImplementedError("write your pallas kernel here")


# speedup vs baseline: 15.9463x; 15.9463x over previous
import numpy as np
import jax
import jax.numpy as jnp
from jax import lax
from jax.experimental import pallas as pl
from jax.experimental.pallas import tpu as pltpu
from jax.experimental.pallas import tpu_sc as plsc

_NUM_LAYERS = 64
_OBS_W = 11
_OBS_H = 11
_GRID = _NUM_LAYERS * _OBS_W * _OBS_H
_BATCH = 4096
_T = 200
_NC, _NS, _L = 2, 16, 16
_NW = _NC * _NS
_BPW = _BATCH // _NW
_NG = (_T + _L - 1) // _L

_FEAT_NORMS = ((0, 1.0), (1, 255.0), (2, 100.0), (3, 30.0), (4, 10.0),
               (5, 255.0), (6, 16.0), (7, 4.0))


def _norm_table():
    t = np.ones(256, np.float32)
    for i, n in _FEAT_NORMS:
        t[i] = n
    return jnp.asarray(t)


def _sc_body(x_hbm, norm_hbm, zero_hbm, out_hbm, xin_v, norm_v, acc_v):
    cid = lax.axis_index("c")
    sid = lax.axis_index("s")
    wid = sid * _NC + cid
    base = wid * _BPW
    pltpu.sync_copy(norm_hbm, norm_v)
    pltpu.sync_copy(zero_hbm, acc_v)
    lanes = lax.iota(jnp.int32, _L)
    fzero = jnp.zeros((_L,), jnp.float32)

    def body(i, carry):
        b = base + i
        pltpu.sync_copy(x_hbm.at[b], xin_v)
        saved = []
        for g in range(_NG):
            tok = lanes + g * _L
            tok_c = jnp.minimum(tok, _T - 1) if g == _NG - 1 else tok
            off = tok_c * 3
            byte = plsc.load_gather(xin_v, [off])
            attr = plsc.load_gather(xin_v, [off + 1])
            val = plsc.load_gather(xin_v, [off + 2])
            attr = jnp.clip(attr, 0, 255)
            norm = plsc.load_gather(norm_v, [attr])
            xc = jnp.bitwise_and(byte, 15)
            yc = lax.shift_right_logical(byte, 4)
            sp = xc * 11 + yc
            wrap = jnp.where(sp > 120, 1, 0)
            s2 = sp - 121 * wrap
            lay = attr + wrap
            xo = lax.shift_right_logical(s2 * 373, 12)
            yo = s2 - xo * 11
            oidx = yo * (_OBS_W * _NUM_LAYERS) + xo * _NUM_LAYERS + lay
            oidx = jnp.where(lay > _NUM_LAYERS - 1, _GRID - 1, oidx)
            ok = byte != 255
            if g == _NG - 1:
                ok = ok & (tok < _T)
            v = jnp.where(ok, val.astype(jnp.float32) / norm, 0.0)
            plsc.addupdate_scatter(acc_v, [oidx], v)
            saved.append(oidx)
        pltpu.sync_copy(acc_v, out_hbm.at[b])
        for oidx in saved:
            plsc.store_scatter(acc_v, [oidx], fzero)
        return carry

    lax.fori_loop(0, _BPW, body, 0)


def kernel(x):
    batch_dims = x.shape[:-2]
    xf = x.reshape(_BATCH, _T * 3)
    mesh = plsc.VectorSubcoreMesh(core_axis_name="c", subcore_axis_name="s",
                                  num_cores=_NC, num_subcores=_NS)
    out = pl.kernel(
        _sc_body,
        out_type=jax.ShapeDtypeStruct((_BATCH, _GRID), jnp.float32),
        mesh=mesh,
        scratch_types=[
            pltpu.VMEM((_T * 3,), jnp.int32),
            pltpu.VMEM((256,), jnp.float32),
            pltpu.VMEM((_GRID,), jnp.float32),
        ],
        compiler_params=pltpu.CompilerParams(needs_layout_passes=False),
    )(xf, _norm_table(), jnp.zeros((_GRID,), jnp.float32))
    return out.reshape(batch_dims + (_OBS_H, _OBS_W, _NUM_LAYERS))
